# Initial kernel scaffold; baseline (speedup 1.0000x reference)
import jax
import jax.numpy as jnp
from jax.experimental import pallas as pl


def kernel(x, edge_index, W1, b1, W2, b2, gamma, beta):
    raise NotImplementedError("write your pallas kernel here")


# speedup vs baseline: 3.4291x; 3.4291x over previous
import functools

import jax
import jax.numpy as jnp
from jax import lax
from jax.experimental import pallas as pl
from jax.experimental.pallas import tpu as pltpu
from jax.experimental.pallas import tpu_sc as plsc

_BN_EPS = 1e-5

_N = 10000
_D = 128
_E = 320000
_NC = 2
_NS = 16
_NW = _NC * _NS
_NPAD = 10240
_RPS = _NPAD // _NS
_CHUNK = 128
_CPT = 80
_EPAD = _NW * _CPT * _CHUNK

_mesh = plsc.VectorSubcoreMesh(core_axis_name="c", subcore_axis_name="s")


@functools.partial(
    pl.kernel,
    mesh=_mesh,
    out_type=jax.ShapeDtypeStruct((_NC, _NPAD, _D), jnp.float32),
    scratch_types=[
        pltpu.VMEM((_CPT, _CHUNK), jnp.int32),
        pltpu.VMEM((_CPT, _CHUNK), jnp.int32),
        pltpu.VMEM((_CHUNK, _D), jnp.float32),
        pltpu.VMEM_SHARED((_NPAD, _D), jnp.float32),
        pltpu.SemaphoreType.DMA,
    ],
)
def _sc_aggregate(x_hbm, src_hbm, dst_hbm, out_hbm,
                  src_v, dst_v, rows_v, agg_sh, sem):
    cid = lax.axis_index("c")
    sid = lax.axis_index("s")
    wid = sid * _NC + cid
    r0 = pl.multiple_of(sid * _RPS, 8)

    pltpu.sync_copy(x_hbm.at[pl.ds(r0, _RPS)], agg_sh.at[pl.ds(r0, _RPS)])
    pltpu.sync_copy(src_hbm.at[wid], src_v)
    pltpu.sync_copy(dst_hbm.at[wid], dst_v)
    plsc.subcore_barrier()

    @pl.loop(0, _CPT)
    def _(j):
        pltpu.async_copy(x_hbm.at[src_v.at[j]], rows_v, sem).wait()
        pltpu.sync_copy(rows_v, agg_sh.at[dst_v.at[j]], add=True)

    plsc.subcore_barrier()
    pltpu.sync_copy(agg_sh.at[pl.ds(r0, _RPS)],
                    out_hbm.at[cid, pl.ds(r0, _RPS)])


def _tc_tail(x, parts, w1, b1, w2, b2, gamma, beta):
    def body(x_ref, p_ref, w1_ref, b1_ref, w2_ref, b2_ref, g_ref, bt_ref,
             o_ref):
        h = p_ref[0, :_N, :] + p_ref[1, :_N, :] - x_ref[...]
        h = jnp.dot(h, w1_ref[...], preferred_element_type=jnp.float32)
        h = jnp.maximum(h + b1_ref[...], 0.0)
        h = jnp.dot(h, w2_ref[...], preferred_element_type=jnp.float32)
        h = h + b2_ref[...]
        mean = jnp.mean(h, axis=0, keepdims=True)
        var = jnp.mean(h * h, axis=0, keepdims=True) - mean * mean
        scale = lax.rsqrt(var + _BN_EPS) * g_ref[...]
        o_ref[...] = jnp.maximum((h - mean) * scale + bt_ref[...], 0.0)

    return pl.pallas_call(
        body,
        out_shape=jax.ShapeDtypeStruct((_N, _D), jnp.float32),
    )(x, parts, w1, b1.reshape(1, _D), w2, b2.reshape(1, _D),
      gamma.reshape(1, _D), beta.reshape(1, _D))


def kernel(x, edge_index, W1, b1, W2, b2, gamma, beta):
    ei = edge_index.astype(jnp.int32)
    pad = _EPAD - _E
    src = jnp.concatenate([ei[0], jnp.zeros((pad,), jnp.int32)])
    dst = jnp.concatenate([ei[1], jnp.full((pad,), _N, jnp.int32)])
    src = src.reshape(_NW, _CPT, _CHUNK)
    dst = dst.reshape(_NW, _CPT, _CHUNK)
    x_pad = jnp.concatenate([x, jnp.zeros((_NPAD - _N, _D), x.dtype)])
    parts = _sc_aggregate(x_pad, src, dst)
    return _tc_tail(x, parts, W1, b1, W2, b2, gamma, beta)


# speedup vs baseline: 3.8766x; 1.1305x over previous
import functools

import jax
import jax.numpy as jnp
from jax import lax
from jax.experimental import pallas as pl
from jax.experimental.pallas import tpu as pltpu
from jax.experimental.pallas import tpu_sc as plsc

_BN_EPS = 1e-5

_N = 10000
_D = 128
_E = 320000
_NC = 2
_NS = 16
_NW = _NC * _NS
_NPAD = 10240
_RPS = _NPAD // _NS
_CHUNK = 128
_CPT = 80
_EPAD = _NW * _CPT * _CHUNK
_NBUF = 2

_mesh = plsc.VectorSubcoreMesh(core_axis_name="c", subcore_axis_name="s")


@functools.partial(
    pl.kernel,
    mesh=_mesh,
    out_type=jax.ShapeDtypeStruct((_NC, _NPAD, _D), jnp.float32),
    scratch_types=[
        pltpu.VMEM((_CPT, _CHUNK), jnp.int32),
        pltpu.VMEM((_NBUF, _CHUNK), jnp.int32),
        pltpu.VMEM((_NBUF, _CHUNK), jnp.int32),
        pltpu.VMEM((_NBUF, _CHUNK, _D), jnp.float32),
        pltpu.VMEM_SHARED((_NPAD, _D), jnp.float32),
    ] + [pltpu.SemaphoreType.DMA] * _NBUF,
)
def _sc_aggregate(x_hbm, combo_hbm, out_hbm,
                  combo_v, sidx_v, didx_v, rows_v, agg_sh, *sems):
    cid = lax.axis_index("c")
    sid = lax.axis_index("s")
    wid = sid * _NC + cid
    r0 = pl.multiple_of(sid * _RPS, 8)

    pltpu.sync_copy(x_hbm.at[pl.ds(r0, _RPS)], agg_sh.at[pl.ds(r0, _RPS)])
    pltpu.sync_copy(combo_hbm.at[wid], combo_v)
    plsc.subcore_barrier()

    def unpack_idx(j, p):
        for k in range(_CHUNK // 16):
            c = combo_v[j, pl.ds(k * 16, 16)]
            sidx_v[p, pl.ds(k * 16, 16)] = lax.shift_right_logical(c, 14)
            didx_v[p, pl.ds(k * 16, 16)] = lax.bitwise_and(c, 16383)

    for p in range(_NBUF):
        unpack_idx(p, p)
        pltpu.async_copy(x_hbm.at[sidx_v.at[p]], rows_v.at[p], sems[p])

    @pl.loop(0, _CPT, step=_NBUF)
    def _(j0):
        for p in range(_NBUF):
            j = j0 + p
            pltpu.make_async_copy(x_hbm.at[sidx_v.at[p]], rows_v.at[p],
                                  sems[p]).wait()
            pltpu.sync_copy(rows_v.at[p], agg_sh.at[didx_v.at[p]], add=True)

            @pl.when(j + _NBUF < _CPT)
            def _():
                unpack_idx(j + _NBUF, p)
                pltpu.async_copy(x_hbm.at[sidx_v.at[p]], rows_v.at[p],
                                 sems[p])

    plsc.subcore_barrier()
    pltpu.sync_copy(agg_sh.at[pl.ds(r0, _RPS)],
                    out_hbm.at[cid, pl.ds(r0, _RPS)])


def _tc_tail(x, parts, w1, b1, w2, b2, gamma, beta):
    def body(x_ref, p_ref, w1_ref, b1_ref, w2_ref, b2_ref, g_ref, bt_ref,
             o_ref):
        h = p_ref[0, :_N, :] + p_ref[1, :_N, :] - x_ref[...]
        h = jnp.dot(h, w1_ref[...], preferred_element_type=jnp.float32)
        h = jnp.maximum(h + b1_ref[...], 0.0)
        h = jnp.dot(h, w2_ref[...], preferred_element_type=jnp.float32)
        h = h + b2_ref[...]
        mean = jnp.mean(h, axis=0, keepdims=True)
        var = jnp.mean(h * h, axis=0, keepdims=True) - mean * mean
        scale = lax.rsqrt(var + _BN_EPS) * g_ref[...]
        o_ref[...] = jnp.maximum((h - mean) * scale + bt_ref[...], 0.0)

    return pl.pallas_call(
        body,
        out_shape=jax.ShapeDtypeStruct((_N, _D), jnp.float32),
    )(x, parts, w1, b1.reshape(1, _D), w2, b2.reshape(1, _D),
      gamma.reshape(1, _D), beta.reshape(1, _D))


def kernel(x, edge_index, W1, b1, W2, b2, gamma, beta):
    ei = edge_index.astype(jnp.int32)
    pad = _EPAD - _E
    combo = jnp.concatenate([
        jnp.left_shift(ei[0], 14) | ei[1],
        jnp.full((pad,), _N, jnp.int32),
    ]).reshape(_NW, _CPT, _CHUNK)
    x_pad = jnp.concatenate([x, jnp.zeros((_NPAD - _N, _D), x.dtype)])
    parts = _sc_aggregate(x_pad, combo)
    return _tc_tail(x, parts, W1, b1, W2, b2, gamma, beta)
